# baseline (speedup 1.0000x reference)
import functools

import jax
import jax.numpy as jnp
from jax import lax
from jax.experimental import pallas as pl
from jax.experimental.pallas import tpu as pltpu
from jax.experimental.pallas import tpu_sc as plsc

NC = 2
NS = 16
NW = NC * NS
N = 65536
D = 512
NB = 50
ROWS_PER_W = N // NW
C = 32
G = ROWS_PER_W // C
MOM = 0.9

_mesh = plsc.VectorSubcoreMesh(core_axis_name="c", subcore_axis_name="s",
                               num_cores=NC, num_subcores=NS)


def _sc_body(feat, lbl, zeros, psum, psq, pcnt, featbuf, lblbuf, acc_s, acc_q,
             acc_c, fsem):
    c = lax.axis_index("c")
    s = lax.axis_index("s")
    wid = s * NC + c
    base0 = wid * ROWS_PER_W

    pltpu.sync_copy(zeros, acc_s)
    pltpu.sync_copy(zeros, acc_q)
    zv = jnp.zeros((16,), jnp.float32)
    for r in range(NB):
        acc_c[r, :] = zv
    pltpu.sync_copy(lbl.at[pl.ds(base0, ROWS_PER_W)], lblbuf)
    lane = jnp.arange(16, dtype=jnp.int32)
    marker = jnp.where(lane == 0, 1.0, 0.0).astype(jnp.float32)

    def fetch(g):
        slot = lax.rem(g, 2)
        return pltpu.async_copy(
            feat.at[pl.ds(base0 + g * C, C)],
            featbuf.at[pl.ds(slot * C, C)],
            fsem.at[slot],
        )

    fetch(0)

    def chunk(g, carry):
        slot = lax.rem(g, 2)
        pltpu.make_async_copy(
            feat.at[pl.ds(base0 + g * C, C)],
            featbuf.at[pl.ds(slot * C, C)],
            fsem.at[slot],
        ).wait()

        @pl.when(g + 1 < G)
        def _():
            fetch(g + 1)

        row0 = slot * C

        def group(gr, carry2):
            lv = lblbuf[pl.ds(g * C + gr * 16, 16)]
            for l in range(16):
                lb = lv[l]
                r = row0 + gr * 16 + l

                for j in range(D // 16):
                    v = featbuf[r, pl.ds(j * 16, 16)]
                    plsc.addupdate(acc_s.at[lb, pl.ds(j * 16, 16)], v)
                    plsc.addupdate(acc_q.at[lb, pl.ds(j * 16, 16)], v * v)
                plsc.addupdate(acc_c.at[lb], marker)
            return carry2

        lax.fori_loop(0, C // 16, group, 0)
        return carry

    lax.fori_loop(0, G, chunk, 0)

    pltpu.sync_copy(acc_s, psum.at[wid])
    pltpu.sync_copy(acc_q, psq.at[wid])
    pltpu.sync_copy(acc_c, pcnt.at[wid])


_sc_call = functools.partial(
    pl.kernel,
    out_type=(
        jax.ShapeDtypeStruct((NW, NB, D), jnp.float32),
        jax.ShapeDtypeStruct((NW, NB, D), jnp.float32),
        jax.ShapeDtypeStruct((NW, NB, 16), jnp.float32),
    ),
    mesh=_mesh,
    scratch_types=[
        pltpu.VMEM((2 * C, D), jnp.float32),
        pltpu.VMEM((ROWS_PER_W,), jnp.int32),
        pltpu.VMEM((NB, D), jnp.float32),
        pltpu.VMEM((NB, D), jnp.float32),
        pltpu.VMEM((NB, 16), jnp.float32),
        pltpu.SemaphoreType.DMA((2,)),
    ],
)(_sc_body)


def _fin_body(ps, pq, pc, rm, rv, nst, om, ov, on):
    sx = jnp.sum(ps[...], axis=0)
    qx = jnp.sum(pq[...], axis=0)
    cnt = jnp.sum(pc[...], axis=0)[:, 0:1]
    safe_n = jnp.maximum(cnt, 1.0)
    mean = sx / safe_n
    denom = jnp.maximum(cnt - 1.0, 1.0)
    var_u = (qx - cnt * mean * mean) / denom
    var_b = qx / safe_n - mean * mean
    var = jnp.where(cnt > 1.0, var_u, var_b)
    present = cnt > 0.0
    om[...] = jnp.where(present, (1.0 - MOM) * mean + MOM * rm[...], rm[...])
    ov[...] = jnp.where(present, (1.0 - MOM) * var + MOM * rv[...], rv[...])
    on[...] = nst[...] + cnt


_fin_call = pl.pallas_call(
    _fin_body,
    out_shape=(
        jax.ShapeDtypeStruct((NB, D), jnp.float32),
        jax.ShapeDtypeStruct((NB, D), jnp.float32),
        jax.ShapeDtypeStruct((NB, 1), jnp.float32),
    ),
)


def kernel(features, labels, running_mean, running_var, num_samples_tracked):
    zeros = jnp.zeros((NB, D), jnp.float32)
    psum, psq, pcnt = _sc_call(features, labels, zeros)
    new_mean, new_var, new_num = _fin_call(
        psum, psq, pcnt, running_mean, running_var,
        num_samples_tracked.reshape(NB, 1))
    return new_mean, new_var, new_num.reshape(NB)


# speedup vs baseline: 1.6747x; 1.6747x over previous
import functools

import jax
import jax.numpy as jnp
from jax import lax
from jax.experimental import pallas as pl
from jax.experimental.pallas import tpu as pltpu
from jax.experimental.pallas import tpu_sc as plsc

NC = 2
NS = 16
NW = NC * NS
N = 65536
D = 512
H = D // 2
NB = 50
NBP = 64
ROWS_PER_W = N // NW
GC = 32
MAXCH = 128
PADIDX = (ROWS_PER_W // GC + NB - 1) * GC + GC
MOM = 0.9

_mesh = plsc.VectorSubcoreMesh(core_axis_name="c", subcore_axis_name="s",
                               num_cores=NC, num_subcores=NS)


def _sc_body(feat, lbl, zeros, psum, psq, pcnt, lblbuf, idxorder, gbuf, acc_s,
             acc_q, cntv, cnt_sm, fill_sm, cbin_sm, cm_sm, csrc_sm, packed_sm,
             fsem):
    c = lax.axis_index("c")
    s = lax.axis_index("s")
    wid = s * NC + c
    base0 = wid * ROWS_PER_W
    lane = jnp.arange(16, dtype=jnp.int32)

    pltpu.sync_copy(zeros, acc_s)
    pltpu.sync_copy(zeros, acc_q)
    pltpu.sync_copy(lbl.at[pl.ds(base0, ROWS_PER_W)], lblbuf)

    def zcnt(b, carry):
        cnt_sm[b] = 0
        return carry

    lax.fori_loop(0, NBP, zcnt, 0)

    def hist(g, carry):
        lv = lblbuf[pl.ds(g * 16, 16)]
        for l in range(16):
            lb = lv[l]
            cnt_sm[lb] = cnt_sm[lb] + 1
        return carry

    lax.fori_loop(0, ROWS_PER_W // 16, hist, 0)

    def bbuild(b, st):
        ci, acc = st
        k = cnt_sm[b]
        fill_sm[b] = acc
        nch_b = lax.div(k + GC - 1, GC)

        def inner(j, ci2, b=b):
            cbin_sm[ci2] = b
            cm_sm[ci2] = jnp.minimum(k - j * GC, GC)
            csrc_sm[ci2] = acc + j * GC
            return ci2 + 1

        ci = lax.fori_loop(0, nch_b, inner, ci)
        return (ci, acc + k)

    nchp, _ = lax.fori_loop(0, NB, bbuild, (jnp.int32(0), jnp.int32(0)))

    def zpack(w, carry):
        packed_sm[w] = 0
        return carry

    lax.fori_loop(0, ROWS_PER_W // 2, zpack, 0)

    def pack(g, carry):
        lv = lblbuf[pl.ds(g * 16, 16)]
        for l in range(16):
            lb = lv[l]
            p = fill_sm[lb]
            fill_sm[lb] = p + 1
            w = lax.shift_right_logical(p, 1)
            sh = lax.bitwise_and(p, 1) * 16
            packed_sm[w] = lax.bitwise_or(
                packed_sm[w], lax.shift_left(g * 16 + l, sh))
        return carry

    lax.fori_loop(0, ROWS_PER_W // 16, pack, 0)

    def unpack(ci, carry):
        src = csrc_sm[ci]
        m = cm_sm[ci]
        for half in range(2):
            vec = jnp.zeros((16,), jnp.int32)
            for l in range(16):
                sl = half * 16 + l
                e = src + sl
                w = lax.shift_right_logical(e, 1)
                sh = lax.bitwise_and(e, 1) * 16
                r = lax.bitwise_and(
                    lax.shift_right_logical(packed_sm[w], sh), 0xFFFF)
                r = jnp.where(sl < m, r, 0)
                vec = jnp.where(lane == l, r, vec)
            idxorder[pl.ds(ci * GC + half * 16, 16)] = vec + base0
        return carry

    lax.fori_loop(0, nchp, unpack, 0)

    def fetch(ci):
        slot = lax.rem(ci, 2)
        return pltpu.async_copy(
            feat.at[idxorder.at[pl.ds(ci * GC, GC)]],
            gbuf.at[pl.ds(slot * GC, GC)],
            fsem.at[slot],
        )

    fetch(0)

    def chunk(ci, carry):
        slot = lax.rem(ci, 2)
        pltpu.make_async_copy(
            feat.at[idxorder.at[pl.ds(ci * GC, GC)]],
            gbuf.at[pl.ds(slot * GC, GC)],
            fsem.at[slot],
        ).wait()

        @pl.when(ci + 1 < nchp)
        def _():
            fetch(ci + 1)

        b = cbin_sm[ci]
        m = cm_sm[ci]
        row0 = slot * GC
        for h in range(2):
            init = tuple(jnp.zeros((16,), jnp.float32) for _ in range(32))

            def rowacc(ri, regs, h=h):
                ss, qq = [], []
                for j in range(16):
                    v = gbuf[row0 + ri, pl.ds(h * H + j * 16, 16)]
                    ss.append(regs[j] + v)
                    qq.append(regs[16 + j] + v * v)
                return tuple(ss) + tuple(qq)

            regs = lax.fori_loop(0, m, rowacc, init)
            for j in range(16):
                plsc.addupdate(acc_s.at[b, pl.ds(h * H + j * 16, 16)],
                               regs[j])
                plsc.addupdate(acc_q.at[b, pl.ds(h * H + j * 16, 16)],
                               regs[16 + j])
        return carry

    lax.fori_loop(0, nchp, chunk, 0)

    for b in range(NBP):
        vecb = jnp.where(lane == 0, cnt_sm[b], 0).astype(jnp.float32)
        cntv[b, :] = vecb
    pltpu.sync_copy(acc_s, psum.at[wid])
    pltpu.sync_copy(acc_q, psq.at[wid])
    pltpu.sync_copy(cntv, pcnt.at[wid])


_sc_call = functools.partial(
    pl.kernel,
    out_type=(
        jax.ShapeDtypeStruct((NW, NB, D), jnp.float32),
        jax.ShapeDtypeStruct((NW, NB, D), jnp.float32),
        jax.ShapeDtypeStruct((NW, NBP, 16), jnp.float32),
    ),
    mesh=_mesh,
    scratch_types=[
        pltpu.VMEM((ROWS_PER_W,), jnp.int32),
        pltpu.VMEM((PADIDX,), jnp.int32),
        pltpu.VMEM((2 * GC, D), jnp.float32),
        pltpu.VMEM((NB, D), jnp.float32),
        pltpu.VMEM((NB, D), jnp.float32),
        pltpu.VMEM((NBP, 16), jnp.float32),
        pltpu.SMEM((NBP,), jnp.int32),
        pltpu.SMEM((NBP,), jnp.int32),
        pltpu.SMEM((MAXCH,), jnp.int32),
        pltpu.SMEM((MAXCH,), jnp.int32),
        pltpu.SMEM((MAXCH,), jnp.int32),
        pltpu.SMEM((ROWS_PER_W // 2,), jnp.int32),
        pltpu.SemaphoreType.DMA((2,)),
    ],
)(_sc_body)


def _fin_body(ps, pq, pc, rm, rv, nst, om, ov, on):
    sx = jnp.sum(ps[...], axis=0)
    qx = jnp.sum(pq[...], axis=0)
    cnt = jnp.sum(pc[...], axis=0)[:NB, 0:1]
    safe_n = jnp.maximum(cnt, 1.0)
    mean = sx / safe_n
    denom = jnp.maximum(cnt - 1.0, 1.0)
    var_u = (qx - cnt * mean * mean) / denom
    var_b = qx / safe_n - mean * mean
    var = jnp.where(cnt > 1.0, var_u, var_b)
    present = cnt > 0.0
    om[...] = jnp.where(present, (1.0 - MOM) * mean + MOM * rm[...], rm[...])
    ov[...] = jnp.where(present, (1.0 - MOM) * var + MOM * rv[...], rv[...])
    on[...] = nst[...] + cnt


_fin_call = pl.pallas_call(
    _fin_body,
    out_shape=(
        jax.ShapeDtypeStruct((NB, D), jnp.float32),
        jax.ShapeDtypeStruct((NB, D), jnp.float32),
        jax.ShapeDtypeStruct((NB, 1), jnp.float32),
    ),
)


def kernel(features, labels, running_mean, running_var, num_samples_tracked):
    zeros = jnp.zeros((NB, D), jnp.float32)
    psum, psq, pcnt = _sc_call(features, labels, zeros)
    new_mean, new_var, new_num = _fin_call(
        psum, psq, pcnt, running_mean, running_var,
        num_samples_tracked.reshape(NB, 1))
    return new_mean, new_var, new_num.reshape(NB)
